# baseline (speedup 1.0000x reference)
import functools

import jax
import jax.numpy as jnp
from jax import lax
from jax.experimental import pallas as pl
from jax.experimental.pallas import tpu as pltpu
from jax.experimental.pallas import tpu_sc as plsc

_B, _S, _H = 4, 8192, 1024
_K, _D = 8192, 64
_N = _B * _S

_NC, _NS = 2, 16
_NW = _NC * _NS
_DP = 128
_ROWS_PER_W = _N // _NW
_SC_CHUNK = 512

_TOK_BLOCK = 2048


def _sc_gather(table_padded, codes_flat):
    mesh = plsc.VectorSubcoreMesh(core_axis_name="c", subcore_axis_name="s")

    @functools.partial(
        pl.kernel,
        mesh=mesh,
        out_type=jax.ShapeDtypeStruct((_N, _DP), jnp.float32),
        scratch_types=[
            pltpu.VMEM((_ROWS_PER_W,), jnp.int32),
            pltpu.VMEM((_SC_CHUNK, _DP), jnp.float32),
            pltpu.SemaphoreType.DMA,
        ],
    )
    def gather_kernel(table_hbm, idx_hbm, out_hbm, idx_v, rows_v, sem):
        wid = lax.axis_index("s") * _NC + lax.axis_index("c")
        base = wid * _ROWS_PER_W
        pltpu.sync_copy(idx_hbm.at[pl.ds(base, _ROWS_PER_W)], idx_v)

        @pl.loop(0, _ROWS_PER_W, step=_SC_CHUNK)
        def _(r):
            pltpu.async_copy(
                table_hbm.at[idx_v.at[pl.ds(r, _SC_CHUNK)]], rows_v, sem
            ).wait()
            pltpu.sync_copy(rows_v, out_hbm.at[pl.ds(base + r, _SC_CHUNK)])

    return gather_kernel(table_padded, codes_flat)


def _tc_body(uncond_ref, embs_ref, par_ref, w_ref, b_ref, g_ref, out_ref):
    lo = embs_ref[:, : _D]
    hi = embs_ref[:, _D:]
    p = par_ref[...]
    e = lo + (hi - lo) * p
    proj = jnp.dot(e.astype(jnp.bfloat16),
                   w_ref[...].astype(jnp.bfloat16),
                   preferred_element_type=jnp.float32)
    out_ref[...] = uncond_ref[...] + (proj + b_ref[...]) * g_ref[...]


def _tc_fused(uncond2d, embs, parity, w, b_proj2d, gate):
    return pl.pallas_call(
        _tc_body,
        grid=(_N // _TOK_BLOCK,),
        in_specs=[
            pl.BlockSpec((_TOK_BLOCK, _H), lambda i: (i, 0)),
            pl.BlockSpec((_TOK_BLOCK, _DP), lambda i: (i, 0)),
            pl.BlockSpec((_TOK_BLOCK, 1), lambda i: (i, 0)),
            pl.BlockSpec((_D, _H), lambda i: (0, 0)),
            pl.BlockSpec((1, _H), lambda i: (0, 0)),
            pl.BlockSpec((1, _H), lambda i: (0, 0)),
        ],
        out_specs=pl.BlockSpec((_TOK_BLOCK, _H), lambda i: (i, 0)),
        out_shape=jax.ShapeDtypeStruct((_N, _H), jnp.float32),
        compiler_params=pltpu.CompilerParams(
            dimension_semantics=("arbitrary",),
        ),
    )(uncond2d, embs, parity, w, b_proj2d, gate)


def kernel(unconditioned, codes, codebook, W_proj, b_proj, gate):
    codes_flat = codes.reshape(_N)
    table_packed = codebook.reshape(_K // 2, _DP)
    idx_packed = jax.lax.shift_right_logical(codes_flat, 1)
    parity = jnp.bitwise_and(codes_flat, 1).astype(jnp.float32).reshape(_N, 1)
    embs = _sc_gather(table_packed, idx_packed)
    uncond2d = unconditioned.reshape(_N, _H)
    out = _tc_fused(uncond2d, embs, parity, W_proj, b_proj.reshape(1, _H),
                    gate)
    return out.reshape(_B, _S, _H)
